# baseline (speedup 1.0000x reference)
import jax
import jax.numpy as jnp
from jax.experimental import pallas as pl
from jax.experimental.pallas import tpu as pltpu

_FEAT = 22
_LANES = 128


def _body(idx_ref, tbl_hbm, out_ref, vbuf, sem):
    i = idx_ref[()]
    col0 = pl.multiple_of((i // _LANES) * _LANES, _LANES)
    copy = pltpu.make_async_copy(
        tbl_hbm.at[:, pl.ds(col0, _LANES)], vbuf, sem
    )
    copy.start()
    copy.wait()
    col = i % _LANES
    x = jnp.transpose(vbuf[...])
    sub = jax.lax.broadcasted_iota(jnp.int32, (_LANES, _FEAT), 0)
    out_ref[...] = jnp.sum(
        jnp.where(sub == col, x, 0.0), axis=0, keepdims=True
    )


_lookup = pl.pallas_call(
    _body,
    in_specs=[
        pl.BlockSpec(memory_space=pltpu.SMEM),
        pl.BlockSpec(memory_space=pl.ANY),
    ],
    out_specs=pl.BlockSpec(memory_space=pltpu.VMEM),
    out_shape=jax.ShapeDtypeStruct((1, _FEAT), jnp.float32),
    scratch_shapes=[
        pltpu.VMEM((_FEAT, _LANES), jnp.float32),
        pltpu.SemaphoreType.DMA,
    ],
    compiler_params=pltpu.CompilerParams(
        skip_device_barrier=True,
        disable_bounds_checks=True,
        disable_semaphore_checks=True,
    ),
)


def kernel(index, table):
    idx = jnp.asarray(index, dtype=jnp.int32)
    tbl_t = pltpu.with_memory_space_constraint(
        table.T, pltpu.MemorySpace.HBM
    )
    return _lookup(idx, tbl_t)
